# baseline (speedup 1.0000x reference)
import functools

import jax
import jax.numpy as jnp
from jax import lax
from jax.experimental import pallas as pl
from jax.experimental.pallas import tpu as pltpu
from jax.experimental.pallas import tpu_sc as plsc

N_ = 10000
E_ = 320000
G_ = 64
DE_ = 16
DG_ = 4
H_ = 128
OUT_ = 2

NC_ = 2
NS_ = 16
NW_ = NC_ * NS_
EW_ = E_ // NW_
CW_ = 40
NT_ = EW_ // CW_
LANE_ = 16

NB_ = 2000
EB_ = 6400

F32 = jnp.float32


@functools.cache
def _sc_mesh():
    return plsc.VectorSubcoreMesh(core_axis_name="c", subcore_axis_name="s",
                                  num_cores=NC_, num_subcores=NS_)


def _relu_sum_chunk(bs, bd, bc):
    def row(r, carry):
        for g in range(H_ // LANE_):
            sl = pl.ds(g * LANE_, LANE_)
            v = bs[r, sl] + bd[r, sl] + bc[r, sl]
            bs[r, sl] = jnp.maximum(v, 0.0)
        return carry
    lax.fori_loop(0, CW_, row, 0)


def _zero_acc_slice(zeros, acc, s):
    rows = 624
    tail = N_ - NS_ * rows
    base = pl.multiple_of(s * rows, 8)
    pltpu.sync_copy(zeros.at[pl.ds(base, rows)], acc.at[pl.ds(base, rows)])

    @pl.when(s == NS_ - 1)
    def _zero_tail():
        pltpu.sync_copy(zeros.at[pl.ds(NS_ * rows, tail)],
                        acc.at[pl.ds(NS_ * rows, tail)])


def _emit_acc_slice(acc, part, s, c):
    rows = 624
    tail = N_ - NS_ * rows
    base = pl.multiple_of(s * rows, 8)
    pltpu.sync_copy(acc.at[pl.ds(base, rows)],
                    part.at[c].at[pl.ds(base, rows)])

    @pl.when(s == NS_ - 1)
    def _out_tail():
        pltpu.sync_copy(acc.at[pl.ds(NS_ * rows, tail)],
                        part.at[c].at[pl.ds(NS_ * rows, tail)])


def _sc_fused_body(write_msg, ts, td, pre, src1, dst1, zeros, *refs):
    if write_msg:
        msg, part = refs[0], refs[1]
        scr = refs[2:]
    else:
        part = refs[0]
        scr = refs[1:]
    (bs0, bd0, bc0, is0, iw0, bs1, bd1, bc1, is1, iw1, acc,
     semi0, semi1, semg0, semg1, semw0, semw1, semc0, semc1) = scr
    slots = ((bs0, bd0, bc0, is0, iw0, semi0, semg0, semw0, semc0),
             (bs1, bd1, bc1, is1, iw1, semi1, semg1, semw1, semc1))

    s = lax.axis_index("s")
    c = lax.axis_index("c")
    wid = s * NC_ + c
    ebase = wid * EW_

    _zero_acc_slice(zeros, acc, s)

    def start_idx(j, slot):
        bs, bd, bc, isx, iw, semi, semg, semw, semc = slot
        off = ebase + j * CW_
        pltpu.async_copy(src1.at[pl.ds(off, CW_)], isx, semi)
        pltpu.async_copy(dst1.at[pl.ds(off, CW_)], iw, semi)
        pltpu.async_copy(pre.at[pl.ds(off, CW_)], bc, semi)

    def start_gather(j, slot):
        bs, bd, bc, isx, iw, semi, semg, semw, semc = slot
        off = ebase + j * CW_
        pltpu.make_async_copy(src1.at[pl.ds(off, CW_)], isx, semi).wait()
        pltpu.make_async_copy(dst1.at[pl.ds(off, CW_)], iw, semi).wait()
        pltpu.make_async_copy(pre.at[pl.ds(off, CW_)], bc, semi).wait()
        pltpu.async_copy(ts.at[isx], bs, semg)
        pltpu.async_copy(td.at[iw], bd, semg)

    def finish(j, slot):
        bs, bd, bc, isx, iw, semi, semg, semw, semc = slot
        pltpu.make_async_copy(ts.at[isx], bs, semg).wait()
        pltpu.make_async_copy(td.at[iw], bd, semg).wait()
        _relu_sum_chunk(bs, bd, bc)
        if write_msg:
            pltpu.async_copy(bs, msg.at[pl.ds(ebase + j * CW_, CW_)], semw)
        pltpu.sync_copy(bs, acc.at[iw], add=True)

    def drain_w(j, slot):
        if write_msg:
            bs = slot[0]
            semw = slot[7]
            pltpu.make_async_copy(bs, msg.at[pl.ds(ebase + j * CW_, CW_)],
                                  semw).wait()

    start_idx(0, slots[0])
    start_idx(1, slots[1])
    start_gather(0, slots[0])
    plsc.subcore_barrier()

    def dbl(k, carry):
        j0 = 2 * k
        j1 = 2 * k + 1
        start_gather(j1, slots[1])
        finish(j0, slots[0])

        @pl.when(j0 + 2 < NT_)
        def _i0():
            start_idx(j0 + 2, slots[0])

        finish(j1, slots[1])
        drain_w(j0, slots[0])

        @pl.when(j1 + 2 < NT_)
        def _i1():
            start_idx(j1 + 2, slots[1])

        drain_w(j1, slots[1])

        @pl.when(j0 + 2 < NT_)
        def _g0():
            start_gather(j0 + 2, slots[0])

        return carry

    lax.fori_loop(0, NT_ // 2, dbl, 0)
    plsc.subcore_barrier()
    _emit_acc_slice(acc, part, s, c)


def _sc_scratch():
    slot = [
        pltpu.VMEM((CW_, H_), F32),
        pltpu.VMEM((CW_, H_), F32),
        pltpu.VMEM((CW_, H_), F32),
        pltpu.VMEM((CW_,), jnp.int32),
        pltpu.VMEM((CW_,), jnp.int32),
    ]
    return slot + slot + [
        pltpu.VMEM_SHARED((N_, H_), F32),
        pltpu.SemaphoreType.DMA,
        pltpu.SemaphoreType.DMA,
        pltpu.SemaphoreType.DMA,
        pltpu.SemaphoreType.DMA,
        pltpu.SemaphoreType.DMA,
        pltpu.SemaphoreType.DMA,
        pltpu.SemaphoreType.DMA,
        pltpu.SemaphoreType.DMA,
    ]


@functools.cache
def _sc_layer_kernel():
    return pl.kernel(
        functools.partial(_sc_fused_body, True),
        out_type=[
            jax.ShapeDtypeStruct((E_, H_), F32),
            jax.ShapeDtypeStruct((NC_, N_, H_), F32),
        ],
        mesh=_sc_mesh(),
        scratch_types=_sc_scratch(),
    )


def _sc_layer0(ts, td, pre, src1, dst1, zeros):
    return _sc_layer_kernel()(ts, td, pre, src1, dst1, zeros)


def _sc_layer1(ts, td, pre, src1, dst1, zeros):
    _, parts = _sc_layer_kernel()(ts, td, pre, src1, dst1, zeros)
    return parts


def _full2d(a, b):
    return pl.BlockSpec((a, b), lambda i: (0, 0))


def _prep_kernel(batch_ref, gl_ref, gm_ref, gsd_ref, wg_ref, bg_ref,
                 wma_ref, wmb_ref, x0_ref, t0s_ref, t0d_ref):
    gl = (gl_ref[...] - gm_ref[...]) / gsd_ref[...]
    genc = jnp.maximum(jnp.dot(gl, wg_ref[...],
                               preferred_element_type=F32) + bg_ref[...], 0.0)
    ids = lax.broadcasted_iota(jnp.int32, (NB_, G_), 1)
    oh = (batch_ref[...] == ids).astype(F32)
    x0 = jnp.dot(oh, genc, preferred_element_type=F32)
    x0_ref[...] = x0
    t0s_ref[...] = jnp.dot(x0, wma_ref[...], preferred_element_type=F32)
    t0d_ref[...] = jnp.dot(x0, wmb_ref[...], preferred_element_type=F32)


def _prep_call(batch2d, gl, gm, gsd, wg, bg, wma, wmb):
    return pl.pallas_call(
        _prep_kernel,
        grid=(N_ // NB_,),
        in_specs=[
            pl.BlockSpec((NB_, 1), lambda i: (i, 0)),
            _full2d(G_, DG_), _full2d(1, DG_), _full2d(1, DG_),
            _full2d(DG_, H_), _full2d(1, H_),
            _full2d(H_, H_), _full2d(H_, H_),
        ],
        out_specs=[
            pl.BlockSpec((NB_, H_), lambda i: (i, 0)),
            pl.BlockSpec((NB_, H_), lambda i: (i, 0)),
            pl.BlockSpec((NB_, H_), lambda i: (i, 0)),
        ],
        out_shape=[jax.ShapeDtypeStruct((N_, H_), F32)] * 3,
    )(batch2d, gl, gm, gsd, wg, bg, wma, wmb)


def _enc_kernel(ea_ref, em_ref, esd_ref, we_ref, be_ref, wc0_ref, bm0_ref,
                t0_ref):
    ea = (ea_ref[...] - em_ref[...]) / esd_ref[...]
    e0 = jnp.maximum(jnp.dot(ea, we_ref[...],
                             preferred_element_type=F32) + be_ref[...], 0.0)
    t0_ref[...] = jnp.dot(e0.astype(jnp.bfloat16),
                          wc0_ref[...].astype(jnp.bfloat16),
                          preferred_element_type=F32) + bm0_ref[...]


def _enc_call(ea, em, esd, we, be, wc0, bm0):
    eb = pl.BlockSpec((EB_, H_), lambda i: (i, 0))
    return pl.pallas_call(
        _enc_kernel,
        grid=(E_ // EB_,),
        in_specs=[
            pl.BlockSpec((EB_, DE_), lambda i: (i, 0)),
            _full2d(1, DE_), _full2d(1, DE_),
            _full2d(DE_, H_), _full2d(1, H_),
            _full2d(H_, H_), _full2d(1, H_),
        ],
        out_specs=eb,
        out_shape=jax.ShapeDtypeStruct((E_, H_), F32),
    )(ea, em, esd, we, be, wc0, bm0)


def _c1_kernel(ea_ref, msg_ref, em_ref, esd_ref, we_ref, be_ref,
               wc1_ref, bm1_ref, c1_ref):
    ea = (ea_ref[...] - em_ref[...]) / esd_ref[...]
    e0 = jnp.maximum(jnp.dot(ea, we_ref[...],
                             preferred_element_type=F32) + be_ref[...], 0.0)
    c1_ref[...] = jnp.dot((e0 + msg_ref[...]).astype(jnp.bfloat16),
                          wc1_ref[...].astype(jnp.bfloat16),
                          preferred_element_type=F32) + bm1_ref[...]


def _c1_call(ea, msg, em, esd, we, be, wc1, bm1):
    eb = pl.BlockSpec((EB_, H_), lambda i: (i, 0))
    return pl.pallas_call(
        _c1_kernel,
        grid=(E_ // EB_,),
        in_specs=[
            pl.BlockSpec((EB_, DE_), lambda i: (i, 0)), eb,
            _full2d(1, DE_), _full2d(1, DE_),
            _full2d(DE_, H_), _full2d(1, H_),
            _full2d(H_, H_), _full2d(1, H_),
        ],
        out_specs=eb,
        out_shape=jax.ShapeDtypeStruct((E_, H_), F32),
    )(ea, msg, em, esd, we, be, wc1, bm1)


def _node0_kernel(x_ref, p0_ref, p1_ref, wnx_ref, wna_ref, bn_ref,
                  wma_ref, wmb_ref, x1_ref, t1s_ref, t1d_ref):
    x = x_ref[...]
    agg = p0_ref[...] + p1_ref[...]
    h = jnp.dot(x, wnx_ref[...], preferred_element_type=F32) + \
        jnp.dot(agg, wna_ref[...], preferred_element_type=F32) + bn_ref[...]
    x1 = x + jnp.maximum(h, 0.0)
    x1_ref[...] = x1
    t1s_ref[...] = jnp.dot(x1, wma_ref[...], preferred_element_type=F32)
    t1d_ref[...] = jnp.dot(x1, wmb_ref[...], preferred_element_type=F32)


def _node0_call(x, p0, p1, wnx, wna, bn, wma, wmb):
    nb = pl.BlockSpec((NB_, H_), lambda i: (i, 0))
    return pl.pallas_call(
        _node0_kernel,
        grid=(N_ // NB_,),
        in_specs=[nb, nb, nb, _full2d(H_, H_), _full2d(H_, H_),
                  _full2d(1, H_), _full2d(H_, H_), _full2d(H_, H_)],
        out_specs=[nb, nb, nb],
        out_shape=[jax.ShapeDtypeStruct((N_, H_), F32)] * 3,
    )(x, p0, p1, wnx, wna, bn, wma, wmb)


def _node1_kernel(x_ref, p0_ref, p1_ref, wnx_ref, wna_ref, bn_ref,
                  wd1_ref, bd1_ref, wd2_ref, bd2_ref, out_ref):
    x = x_ref[...]
    agg = p0_ref[...] + p1_ref[...]
    h = jnp.dot(x, wnx_ref[...], preferred_element_type=F32) + \
        jnp.dot(agg, wna_ref[...], preferred_element_type=F32) + bn_ref[...]
    x2 = x + jnp.maximum(h, 0.0)
    hd = jnp.maximum(jnp.dot(x2, wd1_ref[...],
                             preferred_element_type=F32) + bd1_ref[...], 0.0)
    out_ref[...] = jnp.dot(hd, wd2_ref[...],
                           preferred_element_type=F32) + bd2_ref[...]


def _node1_call(x, p0, p1, wnx, wna, bn, wd1, bd1, wd2, bd2):
    nb = pl.BlockSpec((NB_, H_), lambda i: (i, 0))
    return pl.pallas_call(
        _node1_kernel,
        grid=(N_ // NB_,),
        in_specs=[nb, nb, nb, _full2d(H_, H_), _full2d(H_, H_),
                  _full2d(1, H_), _full2d(H_, H_), _full2d(1, H_),
                  _full2d(H_, OUT_), _full2d(1, OUT_)],
        out_specs=pl.BlockSpec((NB_, OUT_), lambda i: (i, 0)),
        out_shape=jax.ShapeDtypeStruct((N_, OUT_), F32),
    )(x, p0, p1, wnx, wna, bn, wd1, bd1, wd2, bd2)


def kernel(edge_attr, globals_feat, batch, edge_index, e_mean, e_std,
           g_mean, g_std, W_enc_e, b_enc_e, W_enc_g, b_enc_g, Wm, bm,
           Wn, bn, Wd1, bd1, Wd2, bd2):
    src1 = edge_index[0]
    dst1 = edge_index[1]
    batch2d = batch.reshape(N_, 1)
    zeros = jnp.zeros((N_, H_), F32)

    em = e_mean.reshape(1, DE_)
    esd = e_std.reshape(1, DE_)
    gm = g_mean.reshape(1, DG_)
    gsd = g_std.reshape(1, DG_)
    be = b_enc_e.reshape(1, H_)
    bg = b_enc_g.reshape(1, H_)

    wma0, wmb0, wmc0 = Wm[0][:H_], Wm[0][H_:2 * H_], Wm[0][2 * H_:]
    wma1, wmb1, wmc1 = Wm[1][:H_], Wm[1][H_:2 * H_], Wm[1][2 * H_:]
    bm0 = bm[0].reshape(1, H_)
    bm1 = bm[1].reshape(1, H_)
    wnx0, wna0 = Wn[0][:H_], Wn[0][H_:]
    wnx1, wna1 = Wn[1][:H_], Wn[1][H_:]
    bn0 = bn[0].reshape(1, H_)
    bn1 = bn[1].reshape(1, H_)

    x0, t0s, t0d = _prep_call(batch2d, globals_feat, gm, gsd,
                              W_enc_g, bg, wma0, wmb0)
    t0 = _enc_call(edge_attr, em, esd, W_enc_e, be, wmc0, bm0)
    msg0, parts0 = _sc_layer0(t0s, t0d, t0, src1, dst1, zeros)
    c1 = _c1_call(edge_attr, msg0, em, esd, W_enc_e, be, wmc1, bm1)
    x1, t1s, t1d = _node0_call(x0, parts0[0], parts0[1],
                               wnx0, wna0, bn0, wma1, wmb1)
    parts1 = _sc_layer1(t1s, t1d, c1, src1, dst1, zeros)
    out = _node1_call(x1, parts1[0], parts1[1], wnx1, wna1, bn1,
                      Wd1, bd1.reshape(1, H_), Wd2, bd2.reshape(1, OUT_))
    return out


# speedup vs baseline: 1.0085x; 1.0085x over previous
import functools

import jax
import jax.numpy as jnp
from jax import lax
from jax.experimental import pallas as pl
from jax.experimental.pallas import tpu as pltpu
from jax.experimental.pallas import tpu_sc as plsc

N_ = 10000
E_ = 320000
G_ = 64
DE_ = 16
DG_ = 4
H_ = 128
OUT_ = 2

NC_ = 2
NS_ = 16
NW_ = NC_ * NS_
EW_ = E_ // NW_
CW_ = 40
NT_ = EW_ // CW_
LANE_ = 16

NB_ = 10000
EB_ = 12800

F32 = jnp.float32


@functools.cache
def _sc_mesh():
    return plsc.VectorSubcoreMesh(core_axis_name="c", subcore_axis_name="s",
                                  num_cores=NC_, num_subcores=NS_)


def _relu_sum_chunk(bs, bd, bc):
    def row(r, carry):
        for g in range(H_ // LANE_):
            sl = pl.ds(g * LANE_, LANE_)
            v = bs[r, sl] + bd[r, sl] + bc[r, sl]
            bs[r, sl] = jnp.maximum(v, 0.0)
        return carry
    lax.fori_loop(0, CW_, row, 0)


def _zero_acc_slice(zeros, acc, s):
    rows = 624
    tail = N_ - NS_ * rows
    base = pl.multiple_of(s * rows, 8)
    pltpu.sync_copy(zeros.at[pl.ds(base, rows)], acc.at[pl.ds(base, rows)])

    @pl.when(s == NS_ - 1)
    def _zero_tail():
        pltpu.sync_copy(zeros.at[pl.ds(NS_ * rows, tail)],
                        acc.at[pl.ds(NS_ * rows, tail)])


def _emit_acc_slice(acc, part, s, c):
    rows = 624
    tail = N_ - NS_ * rows
    base = pl.multiple_of(s * rows, 8)
    pltpu.sync_copy(acc.at[pl.ds(base, rows)],
                    part.at[c].at[pl.ds(base, rows)])

    @pl.when(s == NS_ - 1)
    def _out_tail():
        pltpu.sync_copy(acc.at[pl.ds(NS_ * rows, tail)],
                        part.at[c].at[pl.ds(NS_ * rows, tail)])


def _sc_fused_body(write_msg, ts, td, pre, src1, dst1, zeros, *refs):
    if write_msg:
        msg, part = refs[0], refs[1]
        scr = refs[2:]
    else:
        part = refs[0]
        scr = refs[1:]
    (bs0, bd0, bc0, is0, iw0, bs1, bd1, bc1, is1, iw1, acc,
     semi0, semi1, semg0, semg1, semw0, semw1, semc0, semc1) = scr
    slots = ((bs0, bd0, bc0, is0, iw0, semi0, semg0, semw0, semc0),
             (bs1, bd1, bc1, is1, iw1, semi1, semg1, semw1, semc1))

    s = lax.axis_index("s")
    c = lax.axis_index("c")
    wid = s * NC_ + c
    ebase = wid * EW_

    _zero_acc_slice(zeros, acc, s)

    def start_idx(j, slot):
        bs, bd, bc, isx, iw, semi, semg, semw, semc = slot
        off = ebase + j * CW_
        pltpu.async_copy(src1.at[pl.ds(off, CW_)], isx, semi)
        pltpu.async_copy(dst1.at[pl.ds(off, CW_)], iw, semi)
        pltpu.async_copy(pre.at[pl.ds(off, CW_)], bc, semi)

    def start_gather(j, slot):
        bs, bd, bc, isx, iw, semi, semg, semw, semc = slot
        off = ebase + j * CW_
        pltpu.make_async_copy(src1.at[pl.ds(off, CW_)], isx, semi).wait()
        pltpu.make_async_copy(dst1.at[pl.ds(off, CW_)], iw, semi).wait()
        pltpu.make_async_copy(pre.at[pl.ds(off, CW_)], bc, semi).wait()
        pltpu.async_copy(ts.at[isx], bs, semg)
        pltpu.async_copy(td.at[iw], bd, semg)

    def finish(j, slot):
        bs, bd, bc, isx, iw, semi, semg, semw, semc = slot
        pltpu.make_async_copy(ts.at[isx], bs, semg).wait()
        pltpu.make_async_copy(td.at[iw], bd, semg).wait()
        _relu_sum_chunk(bs, bd, bc)
        if write_msg:
            pltpu.async_copy(bs, msg.at[pl.ds(ebase + j * CW_, CW_)], semw)
        pltpu.sync_copy(bs, acc.at[iw], add=True)

    def drain_w(j, slot):
        if write_msg:
            bs = slot[0]
            semw = slot[7]
            pltpu.make_async_copy(bs, msg.at[pl.ds(ebase + j * CW_, CW_)],
                                  semw).wait()

    start_idx(0, slots[0])
    start_idx(1, slots[1])
    start_gather(0, slots[0])
    plsc.subcore_barrier()

    def dbl(k, carry):
        j0 = 2 * k
        j1 = 2 * k + 1
        start_gather(j1, slots[1])
        finish(j0, slots[0])

        @pl.when(j0 + 2 < NT_)
        def _i0():
            start_idx(j0 + 2, slots[0])

        finish(j1, slots[1])
        drain_w(j0, slots[0])

        @pl.when(j1 + 2 < NT_)
        def _i1():
            start_idx(j1 + 2, slots[1])

        drain_w(j1, slots[1])

        @pl.when(j0 + 2 < NT_)
        def _g0():
            start_gather(j0 + 2, slots[0])

        return carry

    lax.fori_loop(0, NT_ // 2, dbl, 0)
    plsc.subcore_barrier()
    _emit_acc_slice(acc, part, s, c)


def _sc_scratch():
    slot = [
        pltpu.VMEM((CW_, H_), F32),
        pltpu.VMEM((CW_, H_), F32),
        pltpu.VMEM((CW_, H_), F32),
        pltpu.VMEM((CW_,), jnp.int32),
        pltpu.VMEM((CW_,), jnp.int32),
    ]
    return slot + slot + [
        pltpu.VMEM_SHARED((N_, H_), F32),
        pltpu.SemaphoreType.DMA,
        pltpu.SemaphoreType.DMA,
        pltpu.SemaphoreType.DMA,
        pltpu.SemaphoreType.DMA,
        pltpu.SemaphoreType.DMA,
        pltpu.SemaphoreType.DMA,
        pltpu.SemaphoreType.DMA,
        pltpu.SemaphoreType.DMA,
    ]


@functools.cache
def _sc_layer_kernel():
    return pl.kernel(
        functools.partial(_sc_fused_body, True),
        out_type=[
            jax.ShapeDtypeStruct((E_, H_), F32),
            jax.ShapeDtypeStruct((NC_, N_, H_), F32),
        ],
        mesh=_sc_mesh(),
        scratch_types=_sc_scratch(),
    )


def _sc_layer0(ts, td, pre, src1, dst1, zeros):
    return _sc_layer_kernel()(ts, td, pre, src1, dst1, zeros)


def _sc_layer1(ts, td, pre, src1, dst1, zeros):
    _, parts = _sc_layer_kernel()(ts, td, pre, src1, dst1, zeros)
    return parts


def _full2d(a, b):
    return pl.BlockSpec((a, b), lambda i: (0, 0))


def _prep_kernel(batch_ref, gl_ref, gm_ref, gsd_ref, wg_ref, bg_ref,
                 wma_ref, wmb_ref, x0_ref, t0s_ref, t0d_ref):
    gl = (gl_ref[...] - gm_ref[...]) / gsd_ref[...]
    genc = jnp.maximum(jnp.dot(gl, wg_ref[...],
                               preferred_element_type=F32) + bg_ref[...], 0.0)
    ids = lax.broadcasted_iota(jnp.int32, (NB_, G_), 1)
    oh = (batch_ref[...] == ids).astype(F32)
    x0 = jnp.dot(oh, genc, preferred_element_type=F32)
    x0_ref[...] = x0
    t0s_ref[...] = jnp.dot(x0, wma_ref[...], preferred_element_type=F32)
    t0d_ref[...] = jnp.dot(x0, wmb_ref[...], preferred_element_type=F32)


def _prep_call(batch2d, gl, gm, gsd, wg, bg, wma, wmb):
    return pl.pallas_call(
        _prep_kernel,
        grid=(N_ // NB_,),
        in_specs=[
            pl.BlockSpec((NB_, 1), lambda i: (i, 0)),
            _full2d(G_, DG_), _full2d(1, DG_), _full2d(1, DG_),
            _full2d(DG_, H_), _full2d(1, H_),
            _full2d(H_, H_), _full2d(H_, H_),
        ],
        out_specs=[
            pl.BlockSpec((NB_, H_), lambda i: (i, 0)),
            pl.BlockSpec((NB_, H_), lambda i: (i, 0)),
            pl.BlockSpec((NB_, H_), lambda i: (i, 0)),
        ],
        out_shape=[jax.ShapeDtypeStruct((N_, H_), F32)] * 3,
    )(batch2d, gl, gm, gsd, wg, bg, wma, wmb)


def _enc_kernel(ea_ref, em_ref, esd_ref, we_ref, be_ref, wc0_ref, bm0_ref,
                t0_ref):
    ea = (ea_ref[...] - em_ref[...]) / esd_ref[...]
    e0 = jnp.maximum(jnp.dot(ea, we_ref[...],
                             preferred_element_type=F32) + be_ref[...], 0.0)
    t0_ref[...] = jnp.dot(e0, wc0_ref[...],
                          preferred_element_type=F32) + bm0_ref[...]


def _enc_call(ea, em, esd, we, be, wc0, bm0):
    eb = pl.BlockSpec((EB_, H_), lambda i: (i, 0))
    return pl.pallas_call(
        _enc_kernel,
        grid=(E_ // EB_,),
        in_specs=[
            pl.BlockSpec((EB_, DE_), lambda i: (i, 0)),
            _full2d(1, DE_), _full2d(1, DE_),
            _full2d(DE_, H_), _full2d(1, H_),
            _full2d(H_, H_), _full2d(1, H_),
        ],
        out_specs=eb,
        out_shape=jax.ShapeDtypeStruct((E_, H_), F32),
    )(ea, em, esd, we, be, wc0, bm0)


def _c1_kernel(ea_ref, msg_ref, em_ref, esd_ref, we_ref, be_ref,
               wc1_ref, bm1_ref, c1_ref):
    ea = (ea_ref[...] - em_ref[...]) / esd_ref[...]
    e0 = jnp.maximum(jnp.dot(ea, we_ref[...],
                             preferred_element_type=F32) + be_ref[...], 0.0)
    c1_ref[...] = jnp.dot(e0 + msg_ref[...], wc1_ref[...],
                          preferred_element_type=F32) + bm1_ref[...]


def _c1_call(ea, msg, em, esd, we, be, wc1, bm1):
    eb = pl.BlockSpec((EB_, H_), lambda i: (i, 0))
    return pl.pallas_call(
        _c1_kernel,
        grid=(E_ // EB_,),
        in_specs=[
            pl.BlockSpec((EB_, DE_), lambda i: (i, 0)), eb,
            _full2d(1, DE_), _full2d(1, DE_),
            _full2d(DE_, H_), _full2d(1, H_),
            _full2d(H_, H_), _full2d(1, H_),
        ],
        out_specs=eb,
        out_shape=jax.ShapeDtypeStruct((E_, H_), F32),
    )(ea, msg, em, esd, we, be, wc1, bm1)


def _node0_kernel(x_ref, p0_ref, p1_ref, wnx_ref, wna_ref, bn_ref,
                  wma_ref, wmb_ref, x1_ref, t1s_ref, t1d_ref):
    x = x_ref[...]
    agg = p0_ref[...] + p1_ref[...]
    h = jnp.dot(x, wnx_ref[...], preferred_element_type=F32) + \
        jnp.dot(agg, wna_ref[...], preferred_element_type=F32) + bn_ref[...]
    x1 = x + jnp.maximum(h, 0.0)
    x1_ref[...] = x1
    t1s_ref[...] = jnp.dot(x1, wma_ref[...], preferred_element_type=F32)
    t1d_ref[...] = jnp.dot(x1, wmb_ref[...], preferred_element_type=F32)


def _node0_call(x, p0, p1, wnx, wna, bn, wma, wmb):
    nb = pl.BlockSpec((NB_, H_), lambda i: (i, 0))
    return pl.pallas_call(
        _node0_kernel,
        grid=(N_ // NB_,),
        in_specs=[nb, nb, nb, _full2d(H_, H_), _full2d(H_, H_),
                  _full2d(1, H_), _full2d(H_, H_), _full2d(H_, H_)],
        out_specs=[nb, nb, nb],
        out_shape=[jax.ShapeDtypeStruct((N_, H_), F32)] * 3,
    )(x, p0, p1, wnx, wna, bn, wma, wmb)


def _node1_kernel(x_ref, p0_ref, p1_ref, wnx_ref, wna_ref, bn_ref,
                  wd1_ref, bd1_ref, wd2_ref, bd2_ref, out_ref):
    x = x_ref[...]
    agg = p0_ref[...] + p1_ref[...]
    h = jnp.dot(x, wnx_ref[...], preferred_element_type=F32) + \
        jnp.dot(agg, wna_ref[...], preferred_element_type=F32) + bn_ref[...]
    x2 = x + jnp.maximum(h, 0.0)
    hd = jnp.maximum(jnp.dot(x2, wd1_ref[...],
                             preferred_element_type=F32) + bd1_ref[...], 0.0)
    out_ref[...] = jnp.dot(hd, wd2_ref[...],
                           preferred_element_type=F32) + bd2_ref[...]


def _node1_call(x, p0, p1, wnx, wna, bn, wd1, bd1, wd2, bd2):
    nb = pl.BlockSpec((NB_, H_), lambda i: (i, 0))
    return pl.pallas_call(
        _node1_kernel,
        grid=(N_ // NB_,),
        in_specs=[nb, nb, nb, _full2d(H_, H_), _full2d(H_, H_),
                  _full2d(1, H_), _full2d(H_, H_), _full2d(1, H_),
                  _full2d(H_, OUT_), _full2d(1, OUT_)],
        out_specs=pl.BlockSpec((NB_, OUT_), lambda i: (i, 0)),
        out_shape=jax.ShapeDtypeStruct((N_, OUT_), F32),
    )(x, p0, p1, wnx, wna, bn, wd1, bd1, wd2, bd2)


def kernel(edge_attr, globals_feat, batch, edge_index, e_mean, e_std,
           g_mean, g_std, W_enc_e, b_enc_e, W_enc_g, b_enc_g, Wm, bm,
           Wn, bn, Wd1, bd1, Wd2, bd2):
    src1 = edge_index[0]
    dst1 = edge_index[1]
    batch2d = batch.reshape(N_, 1)
    zeros = jnp.zeros((N_, H_), F32)

    em = e_mean.reshape(1, DE_)
    esd = e_std.reshape(1, DE_)
    gm = g_mean.reshape(1, DG_)
    gsd = g_std.reshape(1, DG_)
    be = b_enc_e.reshape(1, H_)
    bg = b_enc_g.reshape(1, H_)

    wma0, wmb0, wmc0 = Wm[0][:H_], Wm[0][H_:2 * H_], Wm[0][2 * H_:]
    wma1, wmb1, wmc1 = Wm[1][:H_], Wm[1][H_:2 * H_], Wm[1][2 * H_:]
    bm0 = bm[0].reshape(1, H_)
    bm1 = bm[1].reshape(1, H_)
    wnx0, wna0 = Wn[0][:H_], Wn[0][H_:]
    wnx1, wna1 = Wn[1][:H_], Wn[1][H_:]
    bn0 = bn[0].reshape(1, H_)
    bn1 = bn[1].reshape(1, H_)

    x0, t0s, t0d = _prep_call(batch2d, globals_feat, gm, gsd,
                              W_enc_g, bg, wma0, wmb0)
    t0 = _enc_call(edge_attr, em, esd, W_enc_e, be, wmc0, bm0)
    msg0, parts0 = _sc_layer0(t0s, t0d, t0, src1, dst1, zeros)
    c1 = _c1_call(edge_attr, msg0, em, esd, W_enc_e, be, wmc1, bm1)
    x1, t1s, t1d = _node0_call(x0, parts0[0], parts0[1],
                               wnx0, wna0, bn0, wma1, wmb1)
    parts1 = _sc_layer1(t1s, t1d, c1, src1, dst1, zeros)
    out = _node1_call(x1, parts1[0], parts1[1], wnx1, wna1, bn1,
                      Wd1, bd1.reshape(1, H_), Wd2, bd2.reshape(1, OUT_))
    return out


# speedup vs baseline: 1.0088x; 1.0002x over previous
import functools

import jax
import jax.numpy as jnp
from jax import lax
from jax.experimental import pallas as pl
from jax.experimental.pallas import tpu as pltpu
from jax.experimental.pallas import tpu_sc as plsc

N_ = 10000
E_ = 320000
G_ = 64
DE_ = 16
DG_ = 4
H_ = 128
OUT_ = 2

NC_ = 2
NS_ = 16
NW_ = NC_ * NS_
EW_ = E_ // NW_
CW_ = 40
NT_ = EW_ // CW_
LANE_ = 16

NB_ = 10000
EB_ = 12800

F32 = jnp.float32


@functools.cache
def _sc_mesh():
    return plsc.VectorSubcoreMesh(core_axis_name="c", subcore_axis_name="s",
                                  num_cores=NC_, num_subcores=NS_)


def _relu_sum_chunk(bs, bd, bc):
    def row(r, carry):
        for g in range(H_ // LANE_):
            sl = pl.ds(g * LANE_, LANE_)
            v = bs[r, sl] + bd[r, sl] + bc[r, sl]
            bs[r, sl] = jnp.maximum(v, 0.0)
        return carry
    lax.fori_loop(0, CW_, row, 0)


def _zero_acc_slice(zeros, acc, s):
    rows = 624
    tail = N_ - NS_ * rows
    base = pl.multiple_of(s * rows, 8)
    pltpu.sync_copy(zeros.at[pl.ds(base, rows)], acc.at[pl.ds(base, rows)])

    @pl.when(s == NS_ - 1)
    def _zero_tail():
        pltpu.sync_copy(zeros.at[pl.ds(NS_ * rows, tail)],
                        acc.at[pl.ds(NS_ * rows, tail)])


def _emit_acc_slice(acc, part, s, c):
    rows = 624
    tail = N_ - NS_ * rows
    base = pl.multiple_of(s * rows, 8)
    pltpu.sync_copy(acc.at[pl.ds(base, rows)],
                    part.at[c].at[pl.ds(base, rows)])

    @pl.when(s == NS_ - 1)
    def _out_tail():
        pltpu.sync_copy(acc.at[pl.ds(NS_ * rows, tail)],
                        part.at[c].at[pl.ds(NS_ * rows, tail)])


def _sc_fused_body(write_msg, ts, td, pre, src1, dst1, zeros, *refs):
    if write_msg:
        msg, part = refs[0], refs[1]
        scr = refs[2:]
    else:
        part = refs[0]
        scr = refs[1:]
    (bs0, bd0, bc0, is0, iw0, bs1, bd1, bc1, is1, iw1, acc,
     semi0, semi1, semg0, semg1, semw0, semw1) = scr
    slots = ((bs0, bd0, bc0, is0, iw0, semi0, semg0, semw0),
             (bs1, bd1, bc1, is1, iw1, semi1, semg1, semw1))

    s = lax.axis_index("s")
    c = lax.axis_index("c")
    wid = s * NC_ + c
    ebase = wid * EW_

    _zero_acc_slice(zeros, acc, s)

    def start_idx(j, slot):
        bs, bd, bc, isx, iw, semi, semg, semw = slot
        off = ebase + j * CW_
        pltpu.async_copy(src1.at[pl.ds(off, CW_)], isx, semi)
        pltpu.async_copy(dst1.at[pl.ds(off, CW_)], iw, semi)
        pltpu.async_copy(pre.at[pl.ds(off, CW_)], bc, semi)

    def start_gather(j, slot):
        bs, bd, bc, isx, iw, semi, semg, semw = slot
        off = ebase + j * CW_
        pltpu.make_async_copy(src1.at[pl.ds(off, CW_)], isx, semi).wait()
        pltpu.make_async_copy(dst1.at[pl.ds(off, CW_)], iw, semi).wait()
        pltpu.make_async_copy(pre.at[pl.ds(off, CW_)], bc, semi).wait()
        pltpu.async_copy(ts.at[isx], bs, semg)
        pltpu.async_copy(td.at[iw], bd, semg)

    def finish(j, slot):
        bs, bd, bc, isx, iw, semi, semg, semw = slot
        pltpu.make_async_copy(ts.at[isx], bs, semg).wait()
        pltpu.make_async_copy(td.at[iw], bd, semg).wait()
        _relu_sum_chunk(bs, bd, bc)
        if write_msg:
            pltpu.async_copy(bs, msg.at[pl.ds(ebase + j * CW_, CW_)], semw)
        pltpu.sync_copy(bs, acc.at[iw], add=True)

    def drain_w(j, slot):
        if write_msg:
            bs = slot[0]
            semw = slot[7]
            pltpu.make_async_copy(bs, msg.at[pl.ds(ebase + j * CW_, CW_)],
                                  semw).wait()

    start_idx(0, slots[0])
    start_idx(1, slots[1])
    start_gather(0, slots[0])
    plsc.subcore_barrier()

    def dbl(k, carry):
        j0 = 2 * k
        j1 = 2 * k + 1
        start_gather(j1, slots[1])
        finish(j0, slots[0])

        @pl.when(j0 + 2 < NT_)
        def _i0():
            start_idx(j0 + 2, slots[0])

        finish(j1, slots[1])
        drain_w(j0, slots[0])

        @pl.when(j1 + 2 < NT_)
        def _i1():
            start_idx(j1 + 2, slots[1])

        drain_w(j1, slots[1])

        @pl.when(j0 + 2 < NT_)
        def _g0():
            start_gather(j0 + 2, slots[0])

        return carry

    lax.fori_loop(0, NT_ // 2, dbl, 0)
    plsc.subcore_barrier()
    _emit_acc_slice(acc, part, s, c)


def _sc_scratch():
    slot = [
        pltpu.VMEM((CW_, H_), F32),
        pltpu.VMEM((CW_, H_), F32),
        pltpu.VMEM((CW_, H_), F32),
        pltpu.VMEM((CW_,), jnp.int32),
        pltpu.VMEM((CW_,), jnp.int32),
    ]
    return slot + slot + [
        pltpu.VMEM_SHARED((N_, H_), F32),
        pltpu.SemaphoreType.DMA,
        pltpu.SemaphoreType.DMA,
        pltpu.SemaphoreType.DMA,
        pltpu.SemaphoreType.DMA,
        pltpu.SemaphoreType.DMA,
        pltpu.SemaphoreType.DMA,
    ]


@functools.cache
def _sc_layer_kernel():
    return pl.kernel(
        functools.partial(_sc_fused_body, True),
        out_type=[
            jax.ShapeDtypeStruct((E_, H_), F32),
            jax.ShapeDtypeStruct((NC_, N_, H_), F32),
        ],
        mesh=_sc_mesh(),
        scratch_types=_sc_scratch(),
    )


def _sc_layer0(ts, td, pre, src1, dst1, zeros):
    return _sc_layer_kernel()(ts, td, pre, src1, dst1, zeros)


def _sc_layer1(ts, td, pre, src1, dst1, zeros):
    _, parts = _sc_layer_kernel()(ts, td, pre, src1, dst1, zeros)
    return parts


def _full2d(a, b):
    return pl.BlockSpec((a, b), lambda i: (0, 0))


def _prep_kernel(batch_ref, gl_ref, gm_ref, gsd_ref, wg_ref, bg_ref,
                 wma_ref, wmb_ref, x0_ref, t0s_ref, t0d_ref):
    gl = (gl_ref[...] - gm_ref[...]) / gsd_ref[...]
    genc = jnp.maximum(jnp.dot(gl, wg_ref[...],
                               preferred_element_type=F32) + bg_ref[...], 0.0)
    ids = lax.broadcasted_iota(jnp.int32, (NB_, G_), 1)
    oh = (batch_ref[...] == ids).astype(F32)
    x0 = jnp.dot(oh, genc, preferred_element_type=F32)
    x0_ref[...] = x0
    t0s_ref[...] = jnp.dot(x0, wma_ref[...], preferred_element_type=F32)
    t0d_ref[...] = jnp.dot(x0, wmb_ref[...], preferred_element_type=F32)


def _prep_call(batch2d, gl, gm, gsd, wg, bg, wma, wmb):
    return pl.pallas_call(
        _prep_kernel,
        grid=(N_ // NB_,),
        in_specs=[
            pl.BlockSpec((NB_, 1), lambda i: (i, 0)),
            _full2d(G_, DG_), _full2d(1, DG_), _full2d(1, DG_),
            _full2d(DG_, H_), _full2d(1, H_),
            _full2d(H_, H_), _full2d(H_, H_),
        ],
        out_specs=[
            pl.BlockSpec((NB_, H_), lambda i: (i, 0)),
            pl.BlockSpec((NB_, H_), lambda i: (i, 0)),
            pl.BlockSpec((NB_, H_), lambda i: (i, 0)),
        ],
        out_shape=[jax.ShapeDtypeStruct((N_, H_), F32)] * 3,
    )(batch2d, gl, gm, gsd, wg, bg, wma, wmb)


def _enc_kernel(ea_ref, em_ref, esd_ref, we_ref, be_ref, wc0_ref, bm0_ref,
                t0_ref):
    ea = (ea_ref[...] - em_ref[...]) / esd_ref[...]
    e0 = jnp.maximum(jnp.dot(ea, we_ref[...],
                             preferred_element_type=F32) + be_ref[...], 0.0)
    t0_ref[...] = jnp.dot(e0, wc0_ref[...],
                          preferred_element_type=F32) + bm0_ref[...]


def _enc_call(ea, em, esd, we, be, wc0, bm0):
    eb = pl.BlockSpec((EB_, H_), lambda i: (i, 0))
    return pl.pallas_call(
        _enc_kernel,
        grid=(E_ // EB_,),
        in_specs=[
            pl.BlockSpec((EB_, DE_), lambda i: (i, 0)),
            _full2d(1, DE_), _full2d(1, DE_),
            _full2d(DE_, H_), _full2d(1, H_),
            _full2d(H_, H_), _full2d(1, H_),
        ],
        out_specs=eb,
        out_shape=jax.ShapeDtypeStruct((E_, H_), F32),
    )(ea, em, esd, we, be, wc0, bm0)


def _c1_kernel(ea_ref, msg_ref, em_ref, esd_ref, we_ref, be_ref,
               wc1_ref, bm1_ref, c1_ref):
    ea = (ea_ref[...] - em_ref[...]) / esd_ref[...]
    e0 = jnp.maximum(jnp.dot(ea, we_ref[...],
                             preferred_element_type=F32) + be_ref[...], 0.0)
    c1_ref[...] = jnp.dot(e0 + msg_ref[...], wc1_ref[...],
                          preferred_element_type=F32) + bm1_ref[...]


def _c1_call(ea, msg, em, esd, we, be, wc1, bm1):
    eb = pl.BlockSpec((EB_, H_), lambda i: (i, 0))
    return pl.pallas_call(
        _c1_kernel,
        grid=(E_ // EB_,),
        in_specs=[
            pl.BlockSpec((EB_, DE_), lambda i: (i, 0)), eb,
            _full2d(1, DE_), _full2d(1, DE_),
            _full2d(DE_, H_), _full2d(1, H_),
            _full2d(H_, H_), _full2d(1, H_),
        ],
        out_specs=eb,
        out_shape=jax.ShapeDtypeStruct((E_, H_), F32),
    )(ea, msg, em, esd, we, be, wc1, bm1)


def _node0_kernel(x_ref, p0_ref, p1_ref, wnx_ref, wna_ref, bn_ref,
                  wma_ref, wmb_ref, x1_ref, t1s_ref, t1d_ref):
    x = x_ref[...]
    agg = p0_ref[...] + p1_ref[...]
    h = jnp.dot(x, wnx_ref[...], preferred_element_type=F32) + \
        jnp.dot(agg, wna_ref[...], preferred_element_type=F32) + bn_ref[...]
    x1 = x + jnp.maximum(h, 0.0)
    x1_ref[...] = x1
    t1s_ref[...] = jnp.dot(x1, wma_ref[...], preferred_element_type=F32)
    t1d_ref[...] = jnp.dot(x1, wmb_ref[...], preferred_element_type=F32)


def _node0_call(x, p0, p1, wnx, wna, bn, wma, wmb):
    nb = pl.BlockSpec((NB_, H_), lambda i: (i, 0))
    return pl.pallas_call(
        _node0_kernel,
        grid=(N_ // NB_,),
        in_specs=[nb, nb, nb, _full2d(H_, H_), _full2d(H_, H_),
                  _full2d(1, H_), _full2d(H_, H_), _full2d(H_, H_)],
        out_specs=[nb, nb, nb],
        out_shape=[jax.ShapeDtypeStruct((N_, H_), F32)] * 3,
    )(x, p0, p1, wnx, wna, bn, wma, wmb)


def _node1_kernel(x_ref, p0_ref, p1_ref, wnx_ref, wna_ref, bn_ref,
                  wd1_ref, bd1_ref, wd2_ref, bd2_ref, out_ref):
    x = x_ref[...]
    agg = p0_ref[...] + p1_ref[...]
    h = jnp.dot(x, wnx_ref[...], preferred_element_type=F32) + \
        jnp.dot(agg, wna_ref[...], preferred_element_type=F32) + bn_ref[...]
    x2 = x + jnp.maximum(h, 0.0)
    hd = jnp.maximum(jnp.dot(x2, wd1_ref[...],
                             preferred_element_type=F32) + bd1_ref[...], 0.0)
    out_ref[...] = jnp.dot(hd, wd2_ref[...],
                           preferred_element_type=F32) + bd2_ref[...]


def _node1_call(x, p0, p1, wnx, wna, bn, wd1, bd1, wd2, bd2):
    nb = pl.BlockSpec((NB_, H_), lambda i: (i, 0))
    return pl.pallas_call(
        _node1_kernel,
        grid=(N_ // NB_,),
        in_specs=[nb, nb, nb, _full2d(H_, H_), _full2d(H_, H_),
                  _full2d(1, H_), _full2d(H_, H_), _full2d(1, H_),
                  _full2d(H_, OUT_), _full2d(1, OUT_)],
        out_specs=pl.BlockSpec((NB_, OUT_), lambda i: (i, 0)),
        out_shape=jax.ShapeDtypeStruct((N_, OUT_), F32),
    )(x, p0, p1, wnx, wna, bn, wd1, bd1, wd2, bd2)


def kernel(edge_attr, globals_feat, batch, edge_index, e_mean, e_std,
           g_mean, g_std, W_enc_e, b_enc_e, W_enc_g, b_enc_g, Wm, bm,
           Wn, bn, Wd1, bd1, Wd2, bd2):
    src1 = edge_index[0]
    dst1 = edge_index[1]
    batch2d = batch.reshape(N_, 1)
    zeros = jnp.zeros((N_, H_), F32)

    em = e_mean.reshape(1, DE_)
    esd = e_std.reshape(1, DE_)
    gm = g_mean.reshape(1, DG_)
    gsd = g_std.reshape(1, DG_)
    be = b_enc_e.reshape(1, H_)
    bg = b_enc_g.reshape(1, H_)

    wma0, wmb0, wmc0 = Wm[0][:H_], Wm[0][H_:2 * H_], Wm[0][2 * H_:]
    wma1, wmb1, wmc1 = Wm[1][:H_], Wm[1][H_:2 * H_], Wm[1][2 * H_:]
    bm0 = bm[0].reshape(1, H_)
    bm1 = bm[1].reshape(1, H_)
    wnx0, wna0 = Wn[0][:H_], Wn[0][H_:]
    wnx1, wna1 = Wn[1][:H_], Wn[1][H_:]
    bn0 = bn[0].reshape(1, H_)
    bn1 = bn[1].reshape(1, H_)

    x0, t0s, t0d = _prep_call(batch2d, globals_feat, gm, gsd,
                              W_enc_g, bg, wma0, wmb0)
    t0 = _enc_call(edge_attr, em, esd, W_enc_e, be, wmc0, bm0)
    msg0, parts0 = _sc_layer0(t0s, t0d, t0, src1, dst1, zeros)
    c1 = _c1_call(edge_attr, msg0, em, esd, W_enc_e, be, wmc1, bm1)
    x1, t1s, t1d = _node0_call(x0, parts0[0], parts0[1],
                               wnx0, wna0, bn0, wma1, wmb1)
    parts1 = _sc_layer1(t1s, t1d, c1, src1, dst1, zeros)
    out = _node1_call(x1, parts1[0], parts1[1], wnx1, wna1, bn1,
                      Wd1, bd1.reshape(1, H_), Wd2, bd2.reshape(1, OUT_))
    return out
